# baseline (speedup 1.0000x reference)
import functools

import jax
import jax.numpy as jnp
from jax import lax
from jax.experimental import pallas as pl
from jax.experimental.pallas import tpu as pltpu
from jax.experimental.pallas import tpu_sc as plsc

_EPS = 1e-5
_T = 4
_NC = 2
_NS = 16
_SBLK = 1024


def _sc_gather(emb, x):
    b, s = x.shape
    _, d = emb.shape
    n = b * s
    nw = _NC * _NS
    b_per_w = n // nw
    half = b_per_w // 2
    per_row = s // b_per_w
    mesh = plsc.VectorSubcoreMesh(core_axis_name="c", subcore_axis_name="s")

    @functools.partial(
        pl.kernel,
        mesh=mesh,
        out_type=jax.ShapeDtypeStruct((n, d), emb.dtype),
        scratch_types=[
            pltpu.VMEM((half,), jnp.int32),
            pltpu.VMEM((half,), jnp.int32),
            pltpu.VMEM((half, d), emb.dtype),
            pltpu.VMEM((half, d), emb.dtype),
            pltpu.SemaphoreType.DMA,
            pltpu.SemaphoreType.DMA,
        ],
    )
    def gather_kernel(table_hbm, idx_hbm, out_hbm,
                      idx_v0, idx_v1, rows_v0, rows_v1, sem0, sem1):
        wid = lax.axis_index("s") * _NC + lax.axis_index("c")
        row = wid // per_row
        col = (wid % per_row) * b_per_w
        base = wid * b_per_w
        pltpu.sync_copy(idx_hbm.at[row, pl.ds(col, half)], idx_v0)
        pltpu.sync_copy(idx_hbm.at[row, pl.ds(col + half, half)], idx_v1)
        g0 = pltpu.async_copy(table_hbm.at[idx_v0], rows_v0, sem0)
        g0.wait()
        g1 = pltpu.async_copy(table_hbm.at[idx_v1], rows_v1, sem0)
        w0 = pltpu.async_copy(rows_v0, out_hbm.at[pl.ds(base, half)], sem1)
        g1.wait()
        w0.wait()
        pltpu.sync_copy(rows_v1, out_hbm.at[pl.ds(base + half, half)])

    return gather_kernel(emb, x)


def _ln_body(g_ref, poe_ref, gam_ref, bet_ref, o_ref):
    v = g_ref[0] + poe_ref[...]
    mean = jnp.mean(v, axis=-1, keepdims=True)
    c = v - mean
    var = jnp.mean(c * c, axis=-1, keepdims=True)
    y = c * lax.rsqrt(var + _EPS) * gam_ref[...] + bet_ref[...]
    for t in range(_T):
        o_ref[t, 0] = y


def _ln_tile(gathered, poe, gamma, beta, b, s, d):
    return pl.pallas_call(
        _ln_body,
        grid=(b, s // _SBLK),
        in_specs=[
            pl.BlockSpec((1, _SBLK, d), lambda i, j: (i, j, 0)),
            pl.BlockSpec((_SBLK, d), lambda i, j: (j, 0)),
            pl.BlockSpec((1, d), lambda i, j: (0, 0)),
            pl.BlockSpec((1, d), lambda i, j: (0, 0)),
        ],
        out_specs=pl.BlockSpec((_T, 1, _SBLK, d), lambda i, j: (0, i, j, 0)),
        out_shape=jax.ShapeDtypeStruct((_T, b, s, d), jnp.float32),
    )(gathered.reshape(b, s, d), poe, gamma.reshape(1, d), beta.reshape(1, d))


def kernel(x, emb, poe, gamma, beta):
    b, s = x.shape
    _, d = emb.shape
    gathered = _sc_gather(emb, x)
    return _ln_tile(gathered, poe, gamma, beta, b, s, d)


# speedup vs baseline: 1.0703x; 1.0703x over previous
import functools

import jax
import jax.numpy as jnp
from jax import lax
from jax.experimental import pallas as pl
from jax.experimental.pallas import tpu as pltpu
from jax.experimental.pallas import tpu_sc as plsc

_EPS = 1e-5
_T = 4
_NC = 2
_NS = 16
_SBLK = 2048


def _sc_gather(emb, x):
    b, s = x.shape
    _, d = emb.shape
    n = b * s
    nw = _NC * _NS
    b_per_w = n // nw
    half = b_per_w // 2
    per_row = s // b_per_w
    mesh = plsc.VectorSubcoreMesh(core_axis_name="c", subcore_axis_name="s")

    @functools.partial(
        pl.kernel,
        mesh=mesh,
        out_type=jax.ShapeDtypeStruct((n, d), emb.dtype),
        scratch_types=[
            pltpu.VMEM((half,), jnp.int32),
            pltpu.VMEM((half,), jnp.int32),
            pltpu.VMEM((half, d), emb.dtype),
            pltpu.VMEM((half, d), emb.dtype),
            pltpu.SemaphoreType.DMA,
            pltpu.SemaphoreType.DMA,
        ],
    )
    def gather_kernel(table_hbm, idx_hbm, out_hbm,
                      idx_v0, idx_v1, rows_v0, rows_v1, sem0, sem1):
        wid = lax.axis_index("s") * _NC + lax.axis_index("c")
        row = wid // per_row
        col = (wid % per_row) * b_per_w
        base = wid * b_per_w
        pltpu.sync_copy(idx_hbm.at[row, pl.ds(col, half)], idx_v0)
        pltpu.sync_copy(idx_hbm.at[row, pl.ds(col + half, half)], idx_v1)
        g0 = pltpu.async_copy(table_hbm.at[idx_v0], rows_v0, sem0)
        g0.wait()
        g1 = pltpu.async_copy(table_hbm.at[idx_v1], rows_v1, sem0)
        w0 = pltpu.async_copy(rows_v0, out_hbm.at[pl.ds(base, half)], sem1)
        g1.wait()
        w0.wait()
        pltpu.sync_copy(rows_v1, out_hbm.at[pl.ds(base + half, half)])

    return gather_kernel(emb, x)


def _ln_body(g_ref, poe_ref, gam_ref, bet_ref, o_ref):
    v = g_ref[0] + poe_ref[...]
    mean = jnp.mean(v, axis=-1, keepdims=True)
    c = v - mean
    var = jnp.mean(c * c, axis=-1, keepdims=True)
    y = c * lax.rsqrt(var + _EPS) * gam_ref[...] + bet_ref[...]
    for t in range(_T):
        o_ref[t, 0] = y


def _ln_tile(gathered, poe, gamma, beta, b, s, d):
    return pl.pallas_call(
        _ln_body,
        grid=(b, s // _SBLK),
        in_specs=[
            pl.BlockSpec((1, _SBLK, d), lambda i, j: (i, j, 0)),
            pl.BlockSpec((_SBLK, d), lambda i, j: (j, 0)),
            pl.BlockSpec((1, d), lambda i, j: (0, 0)),
            pl.BlockSpec((1, d), lambda i, j: (0, 0)),
        ],
        out_specs=pl.BlockSpec((_T, 1, _SBLK, d), lambda i, j: (0, i, j, 0)),
        out_shape=jax.ShapeDtypeStruct((_T, b, s, d), jnp.float32),
    )(gathered.reshape(b, s, d), poe, gamma.reshape(1, d), beta.reshape(1, d))


def kernel(x, emb, poe, gamma, beta):
    b, s = x.shape
    _, d = emb.shape
    gathered = _sc_gather(emb, x)
    return _ln_tile(gathered, poe, gamma, beta, b, s, d)


# speedup vs baseline: 1.1391x; 1.0643x over previous
import functools

import jax
import jax.numpy as jnp
from jax import lax
from jax.experimental import pallas as pl
from jax.experimental.pallas import tpu as pltpu
from jax.experimental.pallas import tpu_sc as plsc

_EPS = 1e-5
_T = 4
_NC = 2
_NS = 16


def _sc_gather(emb, x):
    b, s = x.shape
    _, d = emb.shape
    n = b * s
    nw = _NC * _NS
    b_per_w = n // nw
    per_row = s // b_per_w
    mesh = plsc.VectorSubcoreMesh(core_axis_name="c", subcore_axis_name="s")

    @functools.partial(
        pl.kernel,
        mesh=mesh,
        out_type=jax.ShapeDtypeStruct((n, d), emb.dtype),
        scratch_types=[
            pltpu.VMEM((b_per_w,), jnp.int32),
            pltpu.VMEM((b_per_w, d), emb.dtype),
            pltpu.SemaphoreType.DMA,
        ],
    )
    def gather_kernel(table_hbm, idx_hbm, out_hbm, idx_v, rows_v, sem):
        wid = lax.axis_index("s") * _NC + lax.axis_index("c")
        row = wid // per_row
        col = (wid % per_row) * b_per_w
        base = wid * b_per_w
        pltpu.sync_copy(idx_hbm.at[row, pl.ds(col, b_per_w)], idx_v)
        pltpu.async_copy(table_hbm.at[idx_v], rows_v, sem).wait()
        pltpu.sync_copy(rows_v, out_hbm.at[pl.ds(base, b_per_w)])

    return gather_kernel(emb, x)


def _ln_body(g_ref, poe_ref, gam_ref, bet_ref, o_hbm, y0, y1, sem0, sem1):
    i = pl.program_id(0)
    n = pl.num_programs(0)
    v = g_ref[0] + poe_ref[...]
    mean = jnp.mean(v, axis=-1, keepdims=True)
    c = v - mean
    var = jnp.mean(c * c, axis=-1, keepdims=True)
    y = c * lax.rsqrt(var + _EPS) * gam_ref[...] + bet_ref[...]

    def issue(scr, sem):
        scr[...] = y
        for t in range(_T):
            pltpu.make_async_copy(scr, o_hbm.at[t, i], sem).start()

    def drain(scr, sem):
        for t in range(_T):
            pltpu.make_async_copy(scr, o_hbm.at[t, i], sem).wait()

    @pl.when(i % 2 == 0)
    def _():
        @pl.when(i >= 2)
        def _():
            drain(y0, sem0)

        issue(y0, sem0)

    @pl.when(i % 2 == 1)
    def _():
        @pl.when(i >= 2)
        def _():
            drain(y1, sem1)

        issue(y1, sem1)

    @pl.when(i == n - 1)
    def _():
        @pl.when(i % 2 == 0)
        def _():
            drain(y0, sem0)
            @pl.when(n >= 2)
            def _():
                drain(y1, sem1)

        @pl.when(i % 2 == 1)
        def _():
            drain(y1, sem1)
            @pl.when(n >= 2)
            def _():
                drain(y0, sem0)


def _ln_tile(gathered, poe, gamma, beta, b, s, d):
    return pl.pallas_call(
        _ln_body,
        grid=(b,),
        in_specs=[
            pl.BlockSpec((1, s, d), lambda i: (i, 0, 0)),
            pl.BlockSpec((s, d), lambda i: (0, 0)),
            pl.BlockSpec((1, d), lambda i: (0, 0)),
            pl.BlockSpec((1, d), lambda i: (0, 0)),
        ],
        out_specs=pl.BlockSpec(memory_space=pltpu.MemorySpace.HBM),
        out_shape=jax.ShapeDtypeStruct((_T, b, s, d), jnp.float32),
        scratch_shapes=[
            pltpu.VMEM((s, d), jnp.float32),
            pltpu.VMEM((s, d), jnp.float32),
            pltpu.SemaphoreType.DMA,
            pltpu.SemaphoreType.DMA,
        ],
    )(gathered.reshape(b, s, d), poe, gamma.reshape(1, d), beta.reshape(1, d))


def kernel(x, emb, poe, gamma, beta):
    b, s = x.shape
    _, d = emb.shape
    gathered = _sc_gather(emb, x)
    return _ln_tile(gathered, poe, gamma, beta, b, s, d)
